# baseline (speedup 1.0000x reference)
import jax
import jax.numpy as jnp
from jax.experimental import pallas as pl

B = 4
N = 1024
K = 20
R = 128
NCH = N // R
M_TOT = float(B * N * K)
INF = float("inf")
BF = jnp.bfloat16
F32 = jnp.float32

_DN_NT = (((1,), (1,)), ((), ()))
_DN_NN = (((1,), (0,)), ((), ()))
_DN_TN = (((0,), (0,)), ((), ()))


def _bdot(a, b, dn):
    return jax.lax.dot_general(a.astype(BF), b.astype(BF), dn,
                               preferred_element_type=F32)


def _bf16_3split(a):
    a0 = a.astype(BF)
    r1 = a - a0.astype(F32)
    a1 = r1.astype(BF)
    a2 = (r1 - a1.astype(F32)).astype(BF)
    return a0, a1, a2


def _knn_body(h_ref, idx_ref):
    h = h_ref[0]
    sq = jnp.sum(h * h, axis=1)
    g = _bdot(h, h, _DN_NT)
    dist = sq[:, None] + sq[None, :] - 2.0 * g
    col = jax.lax.broadcasted_iota(jnp.int32, (N, N), 1)
    d = dist
    idx_rows = []
    for _ in range(K):
        m = jnp.min(d, axis=1, keepdims=True)
        amin = jnp.min(jnp.where(d == m, col, N), axis=1)
        d = jnp.where(col == amin[:, None], INF, d)
        idx_rows.append(amin)
    idx_ref[0] = jnp.stack(idx_rows, axis=0)


def _knn(h):
    c_in = h.shape[-1]
    return pl.pallas_call(
        _knn_body,
        grid=(B,),
        in_specs=[pl.BlockSpec((1, N, c_in), lambda b: (b, 0, 0))],
        out_specs=pl.BlockSpec((1, K, N), lambda b: (b, 0, 0)),
        out_shape=jax.ShapeDtypeStruct((B, K, N), jnp.int32),
    )(h)


def _edge_conv(h_ref, idx_ref, w_ref, j):
    h = h_ref[0]
    idx_blk = idx_ref[0]
    row = jax.lax.broadcasted_iota(jnp.int32, (N, R), 0)
    e_cols = []
    for k in range(K):
        e_cols.append((row == idx_blk[k][None, :]).astype(BF))
    e_t = jnp.concatenate(e_cols, axis=1)
    h0, h1, h2 = _bf16_3split(h)
    dot = lambda u, v: jax.lax.dot_general(u, v, _DN_TN,
                                           preferred_element_type=F32)
    gj = dot(e_t, h0) + dot(e_t, h1) + dot(e_t, h2)
    hc = h_ref[0, pl.ds(j * R, R), :]
    hc_rep = jnp.concatenate([hc] * K, axis=0)
    gf = jnp.concatenate([gj - hc_rep, hc_rep], axis=1)
    return _bdot(gf, w_ref[...], _DN_NN)


def _stats_body(h_ref, idx_ref, w_ref, s1_ref, s2_ref, hh_ref):
    j = pl.program_id(1)
    hh = _edge_conv(h_ref, idx_ref, w_ref, j)
    hh_ref[0, 0] = hh

    @pl.when(j == 0)
    def _():
        s1_ref[...] = jnp.zeros(s1_ref.shape, F32)
        s2_ref[...] = jnp.zeros(s2_ref.shape, F32)

    s1_ref[0, 0, :] += jnp.sum(hh, axis=0)
    s2_ref[0, 0, :] += jnp.sum(hh * hh, axis=0)


def _stats(h, idx, w):
    c_in = h.shape[-1]
    c_out = w.shape[-1]
    return pl.pallas_call(
        _stats_body,
        grid=(B, NCH),
        in_specs=[
            pl.BlockSpec((1, N, c_in), lambda b, j: (b, 0, 0)),
            pl.BlockSpec((1, K, R), lambda b, j: (b, 0, j)),
            pl.BlockSpec((2 * c_in, c_out), lambda b, j: (0, 0)),
        ],
        out_specs=[
            pl.BlockSpec((1, 1, c_out), lambda b, j: (b, 0, 0)),
            pl.BlockSpec((1, 1, c_out), lambda b, j: (b, 0, 0)),
            pl.BlockSpec((1, 1, K * R, c_out), lambda b, j: (b, j, 0, 0)),
        ],
        out_shape=[
            jax.ShapeDtypeStruct((B, 1, c_out), F32),
            jax.ShapeDtypeStruct((B, 1, c_out), F32),
            jax.ShapeDtypeStruct((B, NCH, K * R, c_out), F32),
        ],
    )(h, idx, w)


def _apply_impl(hh_ref, s1_ref, s2_ref, gam_ref, bet_ref,
                we_ref, out_ref):
    hh = hh_ref[0, 0]
    mu = jnp.sum(s1_ref[...], axis=(0, 1)) / M_TOT
    e2 = jnp.sum(s2_ref[...], axis=(0, 1)) / M_TOT
    var = e2 - mu * mu
    rs = jax.lax.rsqrt(var + 1e-5)
    a = gam_ref[...] * rs
    c_out = hh.shape[1]
    acc = jnp.zeros((R, c_out), F32)
    for k in range(K):
        y = (hh[k * R:(k + 1) * R] - mu) * a + bet_ref[...]
        acc = acc + jnp.maximum(y, 0.2 * y)
    h_node = acc * (1.0 / K)
    if we_ref is None:
        out_ref[0] = h_node
    else:
        out_ref[0] = _bdot(h_node, we_ref[...], _DN_NN)


def _apply_plain(hh_ref, s1_ref, s2_ref, gam_ref, bet_ref, out_ref):
    _apply_impl(hh_ref, s1_ref, s2_ref, gam_ref, bet_ref, None, out_ref)


def _apply_fin(hh_ref, s1_ref, s2_ref, gam_ref, bet_ref, we_ref, out_ref):
    _apply_impl(hh_ref, s1_ref, s2_ref, gam_ref, bet_ref, we_ref, out_ref)


def _apply(hh, s1, s2, gam, bet, we=None):
    c_out = hh.shape[-1]
    c_fin = c_out if we is None else we.shape[-1]
    body = _apply_plain if we is None else _apply_fin
    in_specs = [
        pl.BlockSpec((1, 1, K * R, c_out), lambda b, j: (b, j, 0, 0)),
        pl.BlockSpec((B, 1, c_out), lambda b, j: (0, 0, 0)),
        pl.BlockSpec((B, 1, c_out), lambda b, j: (0, 0, 0)),
        pl.BlockSpec((c_out,), lambda b, j: (0,)),
        pl.BlockSpec((c_out,), lambda b, j: (0,)),
    ]
    args = [hh, s1, s2, gam, bet]
    if we is not None:
        in_specs.append(pl.BlockSpec((c_out, c_fin), lambda b, j: (0, 0)))
        args.append(we)
    return pl.pallas_call(
        body,
        grid=(B, NCH),
        in_specs=in_specs,
        out_specs=pl.BlockSpec((1, R, c_fin), lambda b, j: (b, j, 0)),
        out_shape=jax.ShapeDtypeStruct((B, N, c_fin), F32),
    )(*args)


def kernel(x, W0, g0, b0, W1, g1, b1, W2, g2, b2, Wfin, gfin, bfin, We):
    h = x
    outs = []
    for W, g, bt in [(W0, g0, b0), (W1, g1, b1), (W2, g2, b2)]:
        idx = _knn(h)
        s1, s2, hh = _stats(h, idx, W)
        h = _apply(hh, s1, s2, g, bt)
        outs.append(h)
    hcat = jnp.concatenate(outs, axis=-1)
    idx = _knn(hcat)
    s1, s2, hh = _stats(hcat, idx, Wfin)
    return _apply(hh, s1, s2, gfin, bfin, we=We)


# speedup vs baseline: 1.1438x; 1.1438x over previous
import functools

import jax
import jax.numpy as jnp
from jax import lax
from jax.experimental import pallas as pl
from jax.experimental.pallas import tpu as pltpu
from jax.experimental.pallas import tpu_sc as plsc

B = 4
N = 1024
K = 20
R = 128
NCH = N // R
M_TOT = float(B * N * K)
INF = float("inf")
BF = jnp.bfloat16
F32 = jnp.float32

_DN_NT = (((1,), (1,)), ((), ()))
_DN_NN = (((1,), (0,)), ((), ()))
_DN_TN = (((0,), (0,)), ((), ()))


def _bdot(a, b, dn):
    return jax.lax.dot_general(a.astype(BF), b.astype(BF), dn,
                               preferred_element_type=F32)


def _bf16_3split(a):
    a0 = a.astype(BF)
    r1 = a - a0.astype(F32)
    a1 = r1.astype(BF)
    a2 = (r1 - a1.astype(F32)).astype(BF)
    return a0, a1, a2


def _knn_body(h_ref, idx_ref):
    h = h_ref[0]
    sq = jnp.sum(h * h, axis=1)
    g = _bdot(h, h, _DN_NT)
    dist = sq[:, None] + sq[None, :] - 2.0 * g
    col = jax.lax.broadcasted_iota(jnp.int32, (N, N), 1)
    d = dist
    idx_rows = []
    for _ in range(K):
        m = jnp.min(d, axis=1, keepdims=True)
        amin = jnp.min(jnp.where(d == m, col, N), axis=1)
        d = jnp.where(col == amin[:, None], INF, d)
        idx_rows.append(amin)
    idx_ref[0] = jnp.stack(idx_rows, axis=0)


def _knn(h):
    c_in = h.shape[-1]
    return pl.pallas_call(
        _knn_body,
        grid=(B,),
        in_specs=[pl.BlockSpec((1, N, c_in), lambda b: (b, 0, 0))],
        out_specs=pl.BlockSpec((1, K, N), lambda b: (b, 0, 0)),
        out_shape=jax.ShapeDtypeStruct((B, K, N), jnp.int32),
    )(h)


def _edge_conv(h_ref, idx_ref, w_ref, j):
    h = h_ref[0]
    idx_blk = idx_ref[0]
    row = jax.lax.broadcasted_iota(jnp.int32, (N, R), 0)
    e_cols = []
    for k in range(K):
        e_cols.append((row == idx_blk[k][None, :]).astype(BF))
    e_t = jnp.concatenate(e_cols, axis=1)
    h0, h1, h2 = _bf16_3split(h)
    dot = lambda u, v: jax.lax.dot_general(u, v, _DN_TN,
                                           preferred_element_type=F32)
    gj = dot(e_t, h0) + dot(e_t, h1) + dot(e_t, h2)
    hc = h_ref[0, pl.ds(j * R, R), :]
    hc_rep = jnp.concatenate([hc] * K, axis=0)
    gf = jnp.concatenate([gj - hc_rep, hc_rep], axis=1)
    return _bdot(gf, w_ref[...], _DN_NN)


def _stats_body(h_ref, idx_ref, w_ref, s1_ref, s2_ref, hh_ref):
    j = pl.program_id(1)
    hh = _edge_conv(h_ref, idx_ref, w_ref, j)
    hh_ref[0, 0] = hh

    @pl.when(j == 0)
    def _():
        s1_ref[...] = jnp.zeros(s1_ref.shape, F32)
        s2_ref[...] = jnp.zeros(s2_ref.shape, F32)

    s1_ref[0, 0, :] += jnp.sum(hh, axis=0)
    s2_ref[0, 0, :] += jnp.sum(hh * hh, axis=0)


def _stats(h, idx, w):
    c_in = h.shape[-1]
    c_out = w.shape[-1]
    return pl.pallas_call(
        _stats_body,
        grid=(B, NCH),
        in_specs=[
            pl.BlockSpec((1, N, c_in), lambda b, j: (b, 0, 0)),
            pl.BlockSpec((1, K, R), lambda b, j: (b, 0, j)),
            pl.BlockSpec((2 * c_in, c_out), lambda b, j: (0, 0)),
        ],
        out_specs=[
            pl.BlockSpec((1, 1, c_out), lambda b, j: (b, 0, 0)),
            pl.BlockSpec((1, 1, c_out), lambda b, j: (b, 0, 0)),
            pl.BlockSpec((1, 1, K * R, c_out), lambda b, j: (b, j, 0, 0)),
        ],
        out_shape=[
            jax.ShapeDtypeStruct((B, 1, c_out), F32),
            jax.ShapeDtypeStruct((B, 1, c_out), F32),
            jax.ShapeDtypeStruct((B, NCH, K * R, c_out), F32),
        ],
    )(h, idx, w)


def _apply_impl(hh_ref, s1_ref, s2_ref, gam_ref, bet_ref,
                we_ref, out_ref):
    hh = hh_ref[0, 0]
    mu = jnp.sum(s1_ref[...], axis=(0, 1)) / M_TOT
    e2 = jnp.sum(s2_ref[...], axis=(0, 1)) / M_TOT
    var = e2 - mu * mu
    rs = jax.lax.rsqrt(var + 1e-5)
    a = gam_ref[...] * rs
    c_out = hh.shape[1]
    acc = jnp.zeros((R, c_out), F32)
    for k in range(K):
        y = (hh[k * R:(k + 1) * R] - mu) * a + bet_ref[...]
        acc = acc + jnp.maximum(y, 0.2 * y)
    h_node = acc * (1.0 / K)
    if we_ref is None:
        out_ref[0] = h_node
    else:
        out_ref[0] = _bdot(h_node, we_ref[...], _DN_NN)


_SC_WORKERS = 32
_SC_ROWS = (B * K * N) // _SC_WORKERS
_SC_CHUNK = 128


def _sc_gather(h_flat, idx_flat, c):
    mesh = plsc.VectorSubcoreMesh(core_axis_name="c", subcore_axis_name="s")

    @functools.partial(
        pl.kernel,
        mesh=mesh,
        out_type=jax.ShapeDtypeStruct((B * K * N, c), F32),
        scratch_types=[
            pltpu.VMEM((_SC_CHUNK,), jnp.int32),
            pltpu.VMEM((_SC_CHUNK, c), F32),
            pltpu.SemaphoreType.DMA,
        ],
    )
    def gather_kernel(h_hbm, idx_hbm, out_hbm, idx_v, rows_v, sem):
        wid = lax.axis_index("s") * 2 + lax.axis_index("c")
        base = wid * _SC_ROWS
        for t in range(_SC_ROWS // _SC_CHUNK):
            off = base + t * _SC_CHUNK
            pltpu.sync_copy(idx_hbm.at[pl.ds(off, _SC_CHUNK)], idx_v)
            pltpu.async_copy(h_hbm.at[idx_v], rows_v, sem).wait()
            pltpu.sync_copy(rows_v, out_hbm.at[pl.ds(off, _SC_CHUNK)])

    return gather_kernel(h_flat, idx_flat)


def _stats_sc_body(gj_ref, h_ref, w_ref, s1_ref, s2_ref, hh_ref):
    j = pl.program_id(1)
    c = h_ref.shape[2]
    gj = gj_ref[0].reshape(K * R, c)
    hc = h_ref[0, pl.ds(j * R, R), :]
    hc_rep = jnp.concatenate([hc] * K, axis=0)
    gf = jnp.concatenate([gj - hc_rep, hc_rep], axis=1)
    hh = _bdot(gf, w_ref[...], _DN_NN)
    hh_ref[0, 0] = hh

    @pl.when(j == 0)
    def _():
        s1_ref[...] = jnp.zeros(s1_ref.shape, F32)
        s2_ref[...] = jnp.zeros(s2_ref.shape, F32)

    s1_ref[0, 0, :] += jnp.sum(hh, axis=0)
    s2_ref[0, 0, :] += jnp.sum(hh * hh, axis=0)


def _stats_sc(gj, h, w):
    c_in = h.shape[-1]
    c_out = w.shape[-1]
    return pl.pallas_call(
        _stats_sc_body,
        grid=(B, NCH),
        in_specs=[
            pl.BlockSpec((1, K, R, c_in), lambda b, j: (b, 0, j, 0)),
            pl.BlockSpec((1, N, c_in), lambda b, j: (b, 0, 0)),
            pl.BlockSpec((2 * c_in, c_out), lambda b, j: (0, 0)),
        ],
        out_specs=[
            pl.BlockSpec((1, 1, c_out), lambda b, j: (b, 0, 0)),
            pl.BlockSpec((1, 1, c_out), lambda b, j: (b, 0, 0)),
            pl.BlockSpec((1, 1, K * R, c_out), lambda b, j: (b, j, 0, 0)),
        ],
        out_shape=[
            jax.ShapeDtypeStruct((B, 1, c_out), F32),
            jax.ShapeDtypeStruct((B, 1, c_out), F32),
            jax.ShapeDtypeStruct((B, NCH, K * R, c_out), F32),
        ],
    )(gj, h, w)


def _apply_plain(hh_ref, s1_ref, s2_ref, gam_ref, bet_ref, out_ref):
    _apply_impl(hh_ref, s1_ref, s2_ref, gam_ref, bet_ref, None, out_ref)


def _apply_fin(hh_ref, s1_ref, s2_ref, gam_ref, bet_ref, we_ref, out_ref):
    _apply_impl(hh_ref, s1_ref, s2_ref, gam_ref, bet_ref, we_ref, out_ref)


def _apply(hh, s1, s2, gam, bet, we=None):
    c_out = hh.shape[-1]
    c_fin = c_out if we is None else we.shape[-1]
    body = _apply_plain if we is None else _apply_fin
    in_specs = [
        pl.BlockSpec((1, 1, K * R, c_out), lambda b, j: (b, j, 0, 0)),
        pl.BlockSpec((B, 1, c_out), lambda b, j: (0, 0, 0)),
        pl.BlockSpec((B, 1, c_out), lambda b, j: (0, 0, 0)),
        pl.BlockSpec((c_out,), lambda b, j: (0,)),
        pl.BlockSpec((c_out,), lambda b, j: (0,)),
    ]
    args = [hh, s1, s2, gam, bet]
    if we is not None:
        in_specs.append(pl.BlockSpec((c_out, c_fin), lambda b, j: (0, 0)))
        args.append(we)
    return pl.pallas_call(
        body,
        grid=(B, NCH),
        in_specs=in_specs,
        out_specs=pl.BlockSpec((1, R, c_fin), lambda b, j: (b, j, 0)),
        out_shape=jax.ShapeDtypeStruct((B, N, c_fin), F32),
    )(*args)


def kernel(x, W0, g0, b0, W1, g1, b1, W2, g2, b2, Wfin, gfin, bfin, We):
    h = x
    outs = []
    offs = (jnp.arange(B, dtype=jnp.int32) * N)[:, None, None]

    def _sc_layer(h, idx, W):
        c = h.shape[-1]
        c_out = W.shape[-1]
        cp = -(-c // 128) * 128
        hp = jnp.pad(h, ((0, 0), (0, 0), (0, cp - c)))
        wp = jnp.zeros((2 * cp, c_out), F32)
        wp = wp.at[:c].set(W[:c]).at[cp:cp + c].set(W[c:])
        gj = _sc_gather(hp.reshape(B * N, cp),
                        (idx + offs).reshape(B * K * N), cp)
        return _stats_sc(gj.reshape(B, K, N, cp), hp, wp)

    for li, (W, g, bt) in enumerate([(W0, g0, b0), (W1, g1, b1),
                                     (W2, g2, b2)]):
        idx = _knn(h)
        if li == 0:
            s1, s2, hh = _stats(h, idx, W)
        else:
            s1, s2, hh = _sc_layer(h, idx, W)
        h = _apply(hh, s1, s2, g, bt)
        outs.append(h)
    hcat = jnp.concatenate(outs, axis=-1)
    idx = _knn(hcat)
    s1, s2, hh = _sc_layer(hcat, idx, Wfin)
    return _apply(hh, s1, s2, gfin, bfin, we=We)


# speedup vs baseline: 1.1755x; 1.0277x over previous
import functools

import jax
import jax.numpy as jnp
from jax import lax
from jax.experimental import pallas as pl
from jax.experimental.pallas import tpu as pltpu
from jax.experimental.pallas import tpu_sc as plsc

B = 4
N = 1024
K = 20
R = 128
NCH = N // R
M_TOT = float(B * N * K)
INF = float("inf")
BF = jnp.bfloat16
F32 = jnp.float32

_DN_NT = (((1,), (1,)), ((), ()))
_DN_NN = (((1,), (0,)), ((), ()))
_DN_TN = (((0,), (0,)), ((), ()))


def _bdot(a, b, dn):
    return jax.lax.dot_general(a.astype(BF), b.astype(BF), dn,
                               preferred_element_type=F32)


def _bf16_3split(a):
    a0 = a.astype(BF)
    r1 = a - a0.astype(F32)
    a1 = r1.astype(BF)
    a2 = (r1 - a1.astype(F32)).astype(BF)
    return a0, a1, a2


def _knn_body(h_ref, idx_ref):
    h = h_ref[0]
    sq = jnp.sum(h * h, axis=1)
    g = _bdot(h, h, _DN_NT)
    dist = sq[:, None] + sq[None, :] - 2.0 * g
    col = jax.lax.broadcasted_iota(jnp.int32, (N, N), 1)
    d = dist
    idx_rows = []
    for _ in range(K):
        m = jnp.min(d, axis=1, keepdims=True)
        amin = jnp.min(jnp.where(d == m, col, N), axis=1)
        d = jnp.where(col == amin[:, None], INF, d)
        idx_rows.append(amin)
    idx_ref[0] = jnp.stack(idx_rows, axis=0)


def _knn(h):
    c_in = h.shape[-1]
    return pl.pallas_call(
        _knn_body,
        grid=(B,),
        in_specs=[pl.BlockSpec((1, N, c_in), lambda b: (b, 0, 0))],
        out_specs=pl.BlockSpec((1, K, N), lambda b: (b, 0, 0)),
        out_shape=jax.ShapeDtypeStruct((B, K, N), jnp.int32),
    )(h)


def _edge_conv(h_ref, idx_ref, w_ref, j):
    h = h_ref[0]
    idx_blk = idx_ref[0]
    row = jax.lax.broadcasted_iota(jnp.int32, (N, R), 0)
    e_cols = []
    for k in range(K):
        e_cols.append((row == idx_blk[k][None, :]).astype(BF))
    e_t = jnp.concatenate(e_cols, axis=1)
    h0, h1, h2 = _bf16_3split(h)
    dot = lambda u, v: jax.lax.dot_general(u, v, _DN_TN,
                                           preferred_element_type=F32)
    gj = dot(e_t, h0) + dot(e_t, h1) + dot(e_t, h2)
    hc = h_ref[0, pl.ds(j * R, R), :]
    hc_rep = jnp.concatenate([hc] * K, axis=0)
    gf = jnp.concatenate([gj - hc_rep, hc_rep], axis=1)
    return _bdot(gf, w_ref[...], _DN_NN)


def _stats_body(h_ref, idx_ref, w_ref, s1_ref, s2_ref, hh_ref):
    j = pl.program_id(1)
    hh = _edge_conv(h_ref, idx_ref, w_ref, j)
    hh_ref[0, 0] = hh

    @pl.when(j == 0)
    def _():
        s1_ref[...] = jnp.zeros(s1_ref.shape, F32)
        s2_ref[...] = jnp.zeros(s2_ref.shape, F32)

    s1_ref[0, 0, :] += jnp.sum(hh, axis=0)
    s2_ref[0, 0, :] += jnp.sum(hh * hh, axis=0)


def _stats(h, idx, w):
    c_in = h.shape[-1]
    c_out = w.shape[-1]
    return pl.pallas_call(
        _stats_body,
        grid=(B, NCH),
        in_specs=[
            pl.BlockSpec((1, N, c_in), lambda b, j: (b, 0, 0)),
            pl.BlockSpec((1, K, R), lambda b, j: (b, 0, j)),
            pl.BlockSpec((2 * c_in, c_out), lambda b, j: (0, 0)),
        ],
        out_specs=[
            pl.BlockSpec((1, 1, c_out), lambda b, j: (b, 0, 0)),
            pl.BlockSpec((1, 1, c_out), lambda b, j: (b, 0, 0)),
            pl.BlockSpec((1, 1, K * R, c_out), lambda b, j: (b, j, 0, 0)),
        ],
        out_shape=[
            jax.ShapeDtypeStruct((B, 1, c_out), F32),
            jax.ShapeDtypeStruct((B, 1, c_out), F32),
            jax.ShapeDtypeStruct((B, NCH, K * R, c_out), F32),
        ],
    )(h, idx, w)


def _apply_impl(hh_ref, s1_ref, s2_ref, gam_ref, bet_ref,
                we_ref, out_ref):
    hh = hh_ref[0, 0]
    mu = jnp.sum(s1_ref[...], axis=(0, 1)) / M_TOT
    e2 = jnp.sum(s2_ref[...], axis=(0, 1)) / M_TOT
    var = e2 - mu * mu
    rs = jax.lax.rsqrt(var + 1e-5)
    a = gam_ref[...] * rs
    c_out = hh.shape[1]
    acc = jnp.zeros((R, c_out), F32)
    for k in range(K):
        y = (hh[k * R:(k + 1) * R] - mu) * a + bet_ref[...]
        acc = acc + jnp.maximum(y, 0.2 * y)
    h_node = acc * (1.0 / K)
    if we_ref is None:
        out_ref[0] = h_node
    else:
        out_ref[0] = _bdot(h_node, we_ref[...], _DN_NN)


_SC_WORKERS = 32
_SC_ROWS = (B * K * N) // _SC_WORKERS
_SC_CHUNK = 128


def _sc_gather(h_flat, idx_flat, c):
    mesh = plsc.VectorSubcoreMesh(core_axis_name="c", subcore_axis_name="s")

    @functools.partial(
        pl.kernel,
        mesh=mesh,
        out_type=jax.ShapeDtypeStruct((B * K * N, c), F32),
        scratch_types=[
            pltpu.VMEM((_SC_CHUNK,), jnp.int32),
            pltpu.VMEM((_SC_CHUNK, c), F32),
            pltpu.SemaphoreType.DMA,
        ],
    )
    def gather_kernel(h_hbm, idx_hbm, out_hbm, idx_v, rows_v, sem):
        wid = lax.axis_index("s") * 2 + lax.axis_index("c")
        base = wid * _SC_ROWS
        for t in range(_SC_ROWS // _SC_CHUNK):
            off = base + t * _SC_CHUNK
            pltpu.sync_copy(idx_hbm.at[pl.ds(off, _SC_CHUNK)], idx_v)
            pltpu.async_copy(h_hbm.at[idx_v], rows_v, sem).wait()
            pltpu.sync_copy(rows_v, out_hbm.at[pl.ds(off, _SC_CHUNK)])

    return gather_kernel(h_flat, idx_flat)


def _stats_sc_body(gj_ref, h_ref, w_ref, s1_ref, s2_ref, hh_ref):
    j = pl.program_id(1)
    c = h_ref.shape[2]
    gj = gj_ref[0].reshape(K * R, c)
    hc = h_ref[0, pl.ds(j * R, R), :]
    hc_rep = jnp.concatenate([hc] * K, axis=0)
    gf = jnp.concatenate([gj - hc_rep, hc_rep], axis=1)
    hh = _bdot(gf, w_ref[...], _DN_NN)
    hh_ref[0, 0] = hh

    @pl.when(j == 0)
    def _():
        s1_ref[...] = jnp.zeros(s1_ref.shape, F32)
        s2_ref[...] = jnp.zeros(s2_ref.shape, F32)

    s1_ref[0, 0, :] += jnp.sum(hh, axis=0)
    s2_ref[0, 0, :] += jnp.sum(hh * hh, axis=0)


def _stats_sc(gj, h, w):
    c_in = h.shape[-1]
    c_out = w.shape[-1]
    return pl.pallas_call(
        _stats_sc_body,
        grid=(B, NCH),
        in_specs=[
            pl.BlockSpec((1, K, R, c_in), lambda b, j: (b, 0, j, 0)),
            pl.BlockSpec((1, N, c_in), lambda b, j: (b, 0, 0)),
            pl.BlockSpec((2 * c_in, c_out), lambda b, j: (0, 0)),
        ],
        out_specs=[
            pl.BlockSpec((1, 1, c_out), lambda b, j: (b, 0, 0)),
            pl.BlockSpec((1, 1, c_out), lambda b, j: (b, 0, 0)),
            pl.BlockSpec((1, 1, K * R, c_out), lambda b, j: (b, j, 0, 0)),
        ],
        out_shape=[
            jax.ShapeDtypeStruct((B, 1, c_out), F32),
            jax.ShapeDtypeStruct((B, 1, c_out), F32),
            jax.ShapeDtypeStruct((B, NCH, K * R, c_out), F32),
        ],
    )(gj, h, w)


def _apply_plain(hh_ref, s1_ref, s2_ref, gam_ref, bet_ref, out_ref):
    _apply_impl(hh_ref, s1_ref, s2_ref, gam_ref, bet_ref, None, out_ref)


def _apply_fin(hh_ref, s1_ref, s2_ref, gam_ref, bet_ref, we_ref, out_ref):
    _apply_impl(hh_ref, s1_ref, s2_ref, gam_ref, bet_ref, we_ref, out_ref)


def _apply(hh, s1, s2, gam, bet, we=None):
    c_out = hh.shape[-1]
    c_fin = c_out if we is None else we.shape[-1]
    body = _apply_plain if we is None else _apply_fin
    in_specs = [
        pl.BlockSpec((1, 1, K * R, c_out), lambda b, j: (b, j, 0, 0)),
        pl.BlockSpec((B, 1, c_out), lambda b, j: (0, 0, 0)),
        pl.BlockSpec((B, 1, c_out), lambda b, j: (0, 0, 0)),
        pl.BlockSpec((c_out,), lambda b, j: (0,)),
        pl.BlockSpec((c_out,), lambda b, j: (0,)),
    ]
    args = [hh, s1, s2, gam, bet]
    if we is not None:
        in_specs.append(pl.BlockSpec((c_out, c_fin), lambda b, j: (0, 0)))
        args.append(we)
    return pl.pallas_call(
        body,
        grid=(B, NCH),
        in_specs=in_specs,
        out_specs=pl.BlockSpec((1, R, c_fin), lambda b, j: (b, j, 0)),
        out_shape=jax.ShapeDtypeStruct((B, N, c_fin), F32),
    )(*args)


def kernel(x, W0, g0, b0, W1, g1, b1, W2, g2, b2, Wfin, gfin, bfin, We):
    h = x
    outs = []
    offs = (jnp.arange(B, dtype=jnp.int32) * N)[:, None, None]

    def _sc_layer(h, idx, W):
        c = h.shape[-1]
        c_out = W.shape[-1]
        cp = -(-c // 128) * 128
        hp = jnp.pad(h, ((0, 0), (0, 0), (0, cp - c)))
        wp = jnp.zeros((2 * cp, c_out), F32)
        wp = wp.at[:c].set(W[:c]).at[cp:cp + c].set(W[c:])
        gj = _sc_gather(hp.reshape(B * N, cp),
                        (idx + offs).reshape(B * K * N), cp)
        return _stats_sc(gj.reshape(B, K, N, cp), hp, wp)

    for li, (W, g, bt) in enumerate([(W0, g0, b0), (W1, g1, b1),
                                     (W2, g2, b2)]):
        idx = _knn(h)
        s1, s2, hh = _sc_layer(h, idx, W)
        h = _apply(hh, s1, s2, g, bt)
        outs.append(h)
    hcat = jnp.concatenate(outs, axis=-1)
    idx = _knn(hcat)
    s1, s2, hh = _sc_layer(hcat, idx, Wfin)
    return _apply(hh, s1, s2, gfin, bfin, we=We)
